# baseline (speedup 1.0000x reference)
import functools

import jax
import jax.numpy as jnp
from jax import lax
from jax.experimental import pallas as pl
from jax.experimental.pallas import tpu as pltpu
from jax.experimental.pallas import tpu_sc as plsc

N = 500000
FEAT = 32
B, R, SR, K = 1, 4096, 24, 8
M = B * R * SR * K
NW = 32
U = 1024
RT = R // 128
NU = SR * RT
UPW = NU // NW
CH = 128
NCH = U // CH
L = 16
XP = 8
FT = FEAT // 8


def _bcast(cam_v, k):
    return plsc.load_gather(cam_v, [jnp.full((L,), k, jnp.int32)])


NTF = N // 128
NTAIL = N - NTF * 128


def _tr_body(embT_hbm, tail_hbm, out_hbm, colbuf, rowbuf, iota_c):
    wid = lax.axis_index("s") * 2 + lax.axis_index("c")
    iota = lax.iota(jnp.int32, L)

    def tile(c):
        for t in range(4):
            pltpu.sync_copy(
                embT_hbm.at[pl.ds(t * 8, 8), pl.ds(c * 128, 128)],
                colbuf.at[pl.ds(t * 8, 8)])
        _tr_vmem(colbuf, rowbuf, iota)
        pltpu.sync_copy(rowbuf, out_hbm.at[pl.ds(c * (128 * FEAT), 128 * FEAT)])

    def step(i, carry):
        tile(wid + i * NW)
        return carry

    lax.fori_loop(0, NTF // NW, step, 0)

    @pl.when(wid < NTF - (NTF // NW) * NW)
    def _():
        tile((NTF // NW) * NW + wid)

    @pl.when(wid == NW - 1)
    def _():
        for t in range(4):
            pltpu.sync_copy(tail_hbm.at[pl.ds(t * 8, 8)],
                            colbuf.at[pl.ds(t * 8, 8)])
        _tr_vmem(colbuf, rowbuf, iota)
        pltpu.sync_copy(rowbuf.at[pl.ds(0, NTAIL * FEAT)],
                        out_hbm.at[pl.ds(NTF * 128 * FEAT, NTAIL * FEAT)])


def _tr_vmem(colbuf, rowbuf, iota):
    def inner(j, carry):
        rloc = j >> 1
        fvec = (j & 1) * L + iota
        g = plsc.load_gather(colbuf, [fvec, jnp.full((L,), rloc, jnp.int32)])
        rowbuf[pl.ds(rloc * FEAT + (j & 1) * L, L)] = g
        return carry

    lax.fori_loop(0, 256, inner, 0)


@jax.jit
def _sc_transpose(embT, tail):
    run = pl.kernel(
        _tr_body,
        out_type=jax.ShapeDtypeStruct((N * FEAT,), jnp.float32),
        mesh=plsc.VectorSubcoreMesh(core_axis_name="c", subcore_axis_name="s"),
        compiler_params=pltpu.CompilerParams(
            needs_layout_passes=False, use_tc_tiling_on_sc=True),
        scratch_types=[
            pltpu.VMEM((FEAT, 128), jnp.float32),
            pltpu.VMEM((128 * FEAT,), jnp.float32),
            pltpu.SMEM((1,), jnp.int32),
        ],
    )
    return run(embT, tail)


def _sc_body(emb_hbm, xyz_hbm, pidx_hbm, cam_hbm,
             feats_hbm, pers_hbm, xyzw_hbm,
             idx_v, emb_v, xyz_v, feats_t, pers_t, xyzw_t, cam_v,
             sem, sem2, sem3):
    wid = lax.axis_index("s") * 2 + lax.axis_index("c")

    pltpu.sync_copy(cam_hbm, cam_v)
    r = [_bcast(cam_v, k + 1) for k in range(9)]
    cpx = _bcast(cam_v, 10)
    cpy = _bcast(cam_v, 11)
    cpz = _bcast(cam_v, 12)
    iota = lax.iota(jnp.int32, L)
    c0i = jnp.full((L,), 0, jnp.int32)
    c1i = jnp.full((L,), 1, jnp.int32)
    c2i = jnp.full((L,), 2, jnp.int32)

    def unit(i, carry):
        u = wid * UPW + i
        sr = u // RT
        c = u % RT
        pltpu.sync_copy(pidx_hbm.at[pl.ds(u * U, U)], idx_v)
        cps = []
        for j in range(NCH):
            sl = pl.ds(j * CH, CH)
            cps.append(
                pltpu.async_copy(emb_hbm.at[idx_v.at[sl]], emb_v.at[sl], sem))
            cps.append(
                pltpu.async_copy(xyz_hbm.at[idx_v.at[sl]], xyz_v.at[sl], sem2))
        for cp in cps:
            cp.wait()

        def xform(v, carry):
            rvec = iota + v * L
            sl16 = pl.ds(v * L, L)
            x = plsc.load_gather(xyz_v, [rvec, c0i])
            y = plsc.load_gather(xyz_v, [rvec, c1i])
            z = plsc.load_gather(xyz_v, [rvec, c2i])
            xyzw_t[0, sl16] = x
            xyzw_t[1, sl16] = y
            xyzw_t[2, sl16] = z
            sx = x - cpx
            sy = y - cpy
            sz = z - cpz
            v0 = r[0] * sx + r[3] * sy + r[6] * sz
            v1 = r[1] * sx + r[4] * sy + r[7] * sz
            v2 = r[2] * sx + r[5] * sy + r[8] * sz
            den = v2 + 1e-9
            pers_t[0, sl16] = v0 / den
            pers_t[1, sl16] = v1 / den
            pers_t[2, sl16] = v2
            return carry

        lax.fori_loop(0, U // L, xform, 0)

        def tpose(q, carry):
            k = q >> 5
            t = (q >> 3) & 3
            fm = q & 7
            col = jnp.full((L,), t * 8 + fm, jnp.int32)
            row0 = k * 128
            dst = k * FT + t
            for j in range(8):
                g = plsc.load_gather(emb_v, [row0 + j * L + iota, col])
                feats_t[dst, pl.ds(fm * 128 + j * L, L)] = g
            return carry

        lax.fori_loop(0, 256, tpose, 0)

        outs = []
        for k in range(K):
            for t in range(FT):
                off = ((sr * K + k) * 128 + t * RT + c) * U
                outs.append(pltpu.async_copy(
                    feats_t.at[k * FT + t], feats_hbm.at[pl.ds(off, U)], sem3))
        for comp in range(3):
            off = ((sr * 3 + comp) * RT + c) * U
            outs.append(pltpu.async_copy(
                pers_t.at[comp], pers_hbm.at[pl.ds(off, U)], sem3))
            outs.append(pltpu.async_copy(
                xyzw_t.at[comp], xyzw_hbm.at[pl.ds(off, U)], sem3))
        for cp in outs:
            cp.wait()
        return carry

    lax.fori_loop(0, UPW, unit, 0)


@jax.jit
def _sc_gather(points_embeding, xyz_pad, pidx_tiles, cam):
    f32 = jnp.float32
    run = pl.kernel(
        _sc_body,
        out_type=(
            jax.ShapeDtypeStruct((M * FEAT,), f32),
            jax.ShapeDtypeStruct((M * 3,), f32),
            jax.ShapeDtypeStruct((M * 3,), f32),
        ),
        mesh=plsc.VectorSubcoreMesh(core_axis_name="c", subcore_axis_name="s"),
        compiler_params=pltpu.CompilerParams(
            needs_layout_passes=False, use_tc_tiling_on_sc=False),
        scratch_types=[
            pltpu.VMEM((U,), jnp.int32),
            pltpu.VMEM((U, FEAT), f32),
            pltpu.VMEM((U, XP), f32),
            pltpu.VMEM((K * FT, 128 * 8), f32),
            pltpu.VMEM((3, U), f32),
            pltpu.VMEM((3, U), f32),
            pltpu.VMEM((L,), f32),
            pltpu.SemaphoreType.DMA,
            pltpu.SemaphoreType.DMA,
            pltpu.SemaphoreType.DMA,
        ],
    )
    return run(points_embeding, xyz_pad, pidx_tiles, cam)


def kernel(xyz, points_embeding, camrotc2w, campos, sample_pidx):
    pidx_tiles = (sample_pidx.reshape(RT, 128, SR, K)
                  .transpose(2, 0, 3, 1).reshape(-1).astype(jnp.int32))
    cam = jnp.concatenate(
        [jnp.zeros((1,), jnp.float32), camrotc2w.reshape(9),
         campos.reshape(3), jnp.zeros((3,), jnp.float32)]).astype(jnp.float32)
    xyz_pad = jnp.pad(xyz, ((0, 0), (0, XP - 3)))
    embT = points_embeding.T
    tail = jnp.pad(points_embeding[NTF * 128:], ((0, 128 - NTAIL), (0, 0))).T
    emb_rm = _sc_transpose(embT, tail).reshape(N, FEAT)
    feats_img, pers_img, xyzw_img = _sc_gather(
        emb_rm, xyz_pad, pidx_tiles, cam)
    feats = (feats_img.reshape(SR, K, FT, RT, 8, 128)
             .transpose(3, 5, 0, 1, 2, 4).reshape(1, R, SR, K, FEAT))
    pers = (pers_img.reshape(SR, 3, RT, K, 128)
            .transpose(2, 4, 0, 3, 1).reshape(1, R, SR, K, 3))
    xyzw = (xyzw_img.reshape(SR, 3, RT, K, 128)
            .transpose(2, 4, 0, 3, 1).reshape(1, R, SR, K, 3))
    sample_pnt_mask = sample_pidx >= 0
    Rw2c = jnp.eye(3, dtype=xyz.dtype)
    return (feats, pers, xyzw, sample_pnt_mask, Rw2c)


# speedup vs baseline: 1.9543x; 1.9543x over previous
import functools

import jax
import jax.numpy as jnp
from jax import lax
from jax.experimental import pallas as pl
from jax.experimental.pallas import tpu as pltpu
from jax.experimental.pallas import tpu_sc as plsc

N = 500000
FEAT = 32
B, R, SR, K = 1, 4096, 24, 8
M = B * R * SR * K
NW = 32
U = 1024
RT = R // 128
NU = SR * RT
UPW = NU // NW
CH = 128
NCH = U // CH
L = 16
XP = 8
FT = FEAT // 8


def _bcast(cam_v, k):
    return plsc.load_gather(cam_v, [jnp.full((L,), k, jnp.int32)])


NTF = N // 128
NTAIL = N - NTF * 128


def _tr_body(embT_hbm, tail_hbm, out_hbm, colbuf, rowbuf, iota_c):
    wid = lax.axis_index("s") * 2 + lax.axis_index("c")
    iota = lax.iota(jnp.int32, L)

    def tile(c):
        for t in range(4):
            pltpu.sync_copy(
                embT_hbm.at[pl.ds(t * 8, 8), pl.ds(c * 128, 128)],
                colbuf.at[pl.ds(t * 8, 8)])
        _tr_vmem(colbuf, rowbuf, iota)
        pltpu.sync_copy(rowbuf, out_hbm.at[pl.ds(c * (128 * FEAT), 128 * FEAT)])

    def step(i, carry):
        tile(wid + i * NW)
        return carry

    lax.fori_loop(0, NTF // NW, step, 0)

    @pl.when(wid < NTF - (NTF // NW) * NW)
    def _():
        tile((NTF // NW) * NW + wid)

    @pl.when(wid == NW - 1)
    def _():
        for t in range(4):
            pltpu.sync_copy(tail_hbm.at[pl.ds(t * 8, 8)],
                            colbuf.at[pl.ds(t * 8, 8)])
        _tr_vmem(colbuf, rowbuf, iota)
        pltpu.sync_copy(rowbuf.at[pl.ds(0, NTAIL * FEAT)],
                        out_hbm.at[pl.ds(NTF * 128 * FEAT, NTAIL * FEAT)])


def _tr_vmem(colbuf, rowbuf, iota):
    def inner(j, carry):
        rloc = j >> 1
        fvec = (j & 1) * L + iota
        g = plsc.load_gather(colbuf, [fvec, jnp.full((L,), rloc, jnp.int32)])
        rowbuf[pl.ds(rloc * FEAT + (j & 1) * L, L)] = g
        return carry

    lax.fori_loop(0, 256, inner, 0)


@jax.jit
def _sc_transpose(embT, tail):
    run = pl.kernel(
        _tr_body,
        out_type=jax.ShapeDtypeStruct((N * FEAT,), jnp.float32),
        mesh=plsc.VectorSubcoreMesh(core_axis_name="c", subcore_axis_name="s"),
        compiler_params=pltpu.CompilerParams(
            needs_layout_passes=False, use_tc_tiling_on_sc=True),
        scratch_types=[
            pltpu.VMEM((FEAT, 128), jnp.float32),
            pltpu.VMEM((128 * FEAT,), jnp.float32),
            pltpu.SMEM((1,), jnp.int32),
        ],
    )
    return run(embT, tail)


def _sc_body(emb_hbm, xyz_hbm, pidx_hbm, cam_hbm,
             feats_hbm, pers_hbm, xyzw_hbm,
             idx_v, emb_v, xyz_v, feats_t, pers_t, xyzw_t, cam_v,
             sem, sem2, sem3):
    wid = lax.axis_index("s") * 2 + lax.axis_index("c")

    pltpu.sync_copy(cam_hbm, cam_v)
    r = [_bcast(cam_v, k + 1) for k in range(9)]
    cpx = _bcast(cam_v, 10)
    cpy = _bcast(cam_v, 11)
    cpz = _bcast(cam_v, 12)
    iota = lax.iota(jnp.int32, L)
    c0i = jnp.full((L,), 0, jnp.int32)
    c1i = jnp.full((L,), 1, jnp.int32)
    c2i = jnp.full((L,), 2, jnp.int32)

    def unit(i, carry):
        u = wid * UPW + i
        sr = u // RT
        c = u % RT
        pltpu.sync_copy(pidx_hbm.at[pl.ds(u * U, U)], idx_v)
        cps = []
        for j in range(NCH):
            sl = pl.ds(j * CH, CH)
            cps.append(
                pltpu.async_copy(emb_hbm.at[idx_v.at[sl]], emb_v.at[sl], sem))
            cps.append(
                pltpu.async_copy(xyz_hbm.at[idx_v.at[sl]], xyz_v.at[sl], sem2))
        for cp in cps:
            cp.wait()

        def xform(v, carry):
            rvec = iota + v * L
            sl16 = pl.ds(v * L, L)
            x = plsc.load_gather(xyz_v, [rvec, c0i])
            y = plsc.load_gather(xyz_v, [rvec, c1i])
            z = plsc.load_gather(xyz_v, [rvec, c2i])
            xyzw_t[0, sl16] = x
            xyzw_t[1, sl16] = y
            xyzw_t[2, sl16] = z
            sx = x - cpx
            sy = y - cpy
            sz = z - cpz
            v0 = r[0] * sx + r[3] * sy + r[6] * sz
            v1 = r[1] * sx + r[4] * sy + r[7] * sz
            v2 = r[2] * sx + r[5] * sy + r[8] * sz
            den = v2 + 1e-9
            pers_t[0, sl16] = v0 / den
            pers_t[1, sl16] = v1 / den
            pers_t[2, sl16] = v2
            return carry

        lax.fori_loop(0, U // L, xform, 0)

        def tpose(q, carry):
            k = q >> 5
            t = (q >> 3) & 3
            fm = q & 7
            col = jnp.full((L,), t * 8 + fm, jnp.int32)
            row0 = k * 128
            dst = k * FT + t
            for j in range(8):
                g = plsc.load_gather(emb_v, [row0 + j * L + iota, col])
                feats_t[dst, pl.ds(fm * 128 + j * L, L)] = g
            return carry

        lax.fori_loop(0, 256, tpose, 0)

        outs = []
        for k in range(K):
            for t in range(FT):
                off = ((sr * K + k) * 128 + t * RT + c) * U
                outs.append(pltpu.async_copy(
                    feats_t.at[k * FT + t], feats_hbm.at[pl.ds(off, U)], sem3))
        for comp in range(3):
            off = ((sr * 3 + comp) * RT + c) * U
            outs.append(pltpu.async_copy(
                pers_t.at[comp], pers_hbm.at[pl.ds(off, U)], sem3))
            outs.append(pltpu.async_copy(
                xyzw_t.at[comp], xyzw_hbm.at[pl.ds(off, U)], sem3))
        for cp in outs:
            cp.wait()
        return carry

    lax.fori_loop(0, UPW, unit, 0)


@jax.jit
def _sc_gather(points_embeding, xyz_pad, pidx_tiles, cam):
    f32 = jnp.float32
    run = pl.kernel(
        _sc_body,
        out_type=(
            jax.ShapeDtypeStruct((M * FEAT,), f32),
            jax.ShapeDtypeStruct((M * 3,), f32),
            jax.ShapeDtypeStruct((M * 3,), f32),
        ),
        mesh=plsc.VectorSubcoreMesh(core_axis_name="c", subcore_axis_name="s"),
        compiler_params=pltpu.CompilerParams(
            needs_layout_passes=False, use_tc_tiling_on_sc=False),
        scratch_types=[
            pltpu.VMEM((U,), jnp.int32),
            pltpu.VMEM((U, FEAT), f32),
            pltpu.VMEM((U, XP), f32),
            pltpu.VMEM((K * FT, 128 * 8), f32),
            pltpu.VMEM((3, U), f32),
            pltpu.VMEM((3, U), f32),
            pltpu.VMEM((L,), f32),
            pltpu.SemaphoreType.DMA,
            pltpu.SemaphoreType.DMA,
            pltpu.SemaphoreType.DMA,
        ],
    )
    return run(points_embeding, xyz_pad, pidx_tiles, cam)


def kernel(xyz, points_embeding, camrotc2w, campos, sample_pidx):
    pidx_tiles = (sample_pidx.reshape(RT, 128, SR, K)
                  .transpose(2, 0, 3, 1).reshape(-1).astype(jnp.int32))
    cam = jnp.concatenate(
        [jnp.zeros((1,), jnp.float32), camrotc2w.reshape(9),
         campos.reshape(3), jnp.zeros((3,), jnp.float32)]).astype(jnp.float32)
    zcol = jnp.zeros((N,), jnp.float32)
    xyz_pad = jnp.stack(
        [xyz[:, 0], xyz[:, 1], xyz[:, 2], zcol, zcol, zcol, zcol, zcol],
        axis=1)
    embT = points_embeding.T
    tail = jnp.pad(points_embeding[NTF * 128:], ((0, 128 - NTAIL), (0, 0))).T
    emb_rm = _sc_transpose(embT, tail).reshape(N, FEAT)
    feats_img, pers_img, xyzw_img = _sc_gather(
        emb_rm, xyz_pad, pidx_tiles, cam)
    feats = (feats_img.reshape(SR, K, FT, RT, 8, 128)
             .transpose(3, 5, 0, 1, 2, 4).reshape(1, R, SR, K, FEAT))
    pers = (pers_img.reshape(SR, 3, RT, K, 128)
            .transpose(2, 4, 0, 3, 1).reshape(1, R, SR, K, 3))
    xyzw = (xyzw_img.reshape(SR, 3, RT, K, 128)
            .transpose(2, 4, 0, 3, 1).reshape(1, R, SR, K, 3))
    sample_pnt_mask = sample_pidx >= 0
    Rw2c = jnp.eye(3, dtype=xyz.dtype)
    return (feats, pers, xyzw, sample_pnt_mask, Rw2c)


# speedup vs baseline: 2.4137x; 1.2351x over previous
import functools

import jax
import jax.numpy as jnp
from jax import lax
from jax.experimental import pallas as pl
from jax.experimental.pallas import tpu as pltpu
from jax.experimental.pallas import tpu_sc as plsc

N = 500000
FEAT = 32
B, R, SR, K = 1, 4096, 24, 8
M = B * R * SR * K
NW = 32
U = 1024
RT = R // 128
NU = SR * RT
UPW = NU // NW
CH = 128
NCH = U // CH
L = 16
XP = 8
FT = FEAT // 8


def _bcast(cam_v, k):
    return plsc.load_gather(cam_v, [jnp.full((L,), k, jnp.int32)])


NTF = N // 128
NTAIL = N - NTF * 128


ST = 1024
NST = (NTF * 128) // ST
STPW = NST // NW
STX = NST - STPW * NW
LEFT = NTF - (NST * ST) // 128


def _tr_body(embT_hbm, tail_hbm, out_hbm, colbuf, rowbuf, semT):
    wid = lax.axis_index("s") * 2 + lax.axis_index("c")
    iota = lax.iota(jnp.int32, L)
    iota_lo = iota * ST
    iota_hi = iota_lo + L * ST

    def transp(npts):
        def inner(rloc, carry):
            g0 = plsc.load_gather(colbuf, [iota_lo + rloc])
            g1 = plsc.load_gather(colbuf, [iota_hi + rloc])
            rowbuf[pl.ds(rloc * FEAT, L)] = g0
            rowbuf[pl.ds(rloc * FEAT + L, L)] = g1
            return carry

        lax.fori_loop(0, npts, inner, 0, unroll=4)

    def block(col0, npts):
        cps = [pltpu.async_copy(
            embT_hbm.at[f, pl.ds(col0, npts)],
            colbuf.at[pl.ds(f * ST, npts)], semT)
            for f in range(FEAT)]
        for cp in cps:
            cp.wait()
        transp(npts)
        pltpu.sync_copy(rowbuf.at[pl.ds(0, npts * FEAT)],
                        out_hbm.at[pl.ds(col0 * FEAT, npts * FEAT)])

    def step(i, carry):
        block((wid + i * NW) * ST, ST)
        return carry

    lax.fori_loop(0, STPW, step, 0)

    @pl.when(wid < STX)
    def _():
        block((STPW * NW + wid) * ST, ST)

    @pl.when((wid >= STX) & (wid < STX + LEFT))
    def _():
        block(NST * ST + (wid - STX) * 128, 128)

    @pl.when(wid == NW - 1)
    def _():
        cps = [pltpu.async_copy(
            tail_hbm.at[f], colbuf.at[pl.ds(f * ST, 128)], semT)
            for f in range(FEAT)]
        for cp in cps:
            cp.wait()
        transp(NTAIL)
        pltpu.sync_copy(rowbuf.at[pl.ds(0, NTAIL * FEAT)],
                        out_hbm.at[pl.ds(NTF * 128 * FEAT, NTAIL * FEAT)])


@jax.jit
def _sc_transpose(embT, tail):
    run = pl.kernel(
        _tr_body,
        out_type=jax.ShapeDtypeStruct((N * FEAT,), jnp.float32),
        mesh=plsc.VectorSubcoreMesh(core_axis_name="c", subcore_axis_name="s"),
        compiler_params=pltpu.CompilerParams(
            needs_layout_passes=False, use_tc_tiling_on_sc=True),
        scratch_types=[
            pltpu.VMEM((FEAT * ST,), jnp.float32),
            pltpu.VMEM((ST * FEAT,), jnp.float32),
            pltpu.SemaphoreType.DMA,
        ],
    )
    return run(embT, tail)


def _sc_body(emb_hbm, xyz_hbm, pidx_hbm, cam_hbm,
             feats_hbm, pers_hbm, xyzw_hbm,
             idx_v, emb_v, xyz_v, feats_t, pers_t, xyzw_t, cam_v,
             sem, sem2, sem3):
    wid = lax.axis_index("s") * 2 + lax.axis_index("c")

    pltpu.sync_copy(cam_hbm, cam_v)
    r = [_bcast(cam_v, k + 1) for k in range(9)]
    cpx = _bcast(cam_v, 10)
    cpy = _bcast(cam_v, 11)
    cpz = _bcast(cam_v, 12)
    iota = lax.iota(jnp.int32, L)
    c0i = jnp.full((L,), 0, jnp.int32)
    c1i = jnp.full((L,), 1, jnp.int32)
    c2i = jnp.full((L,), 2, jnp.int32)

    def unit(i, carry):
        u = wid * UPW + i
        sr = u // RT
        c = u % RT
        pltpu.sync_copy(pidx_hbm.at[pl.ds(u * U, U)], idx_v)
        cps = []
        for j in range(NCH):
            sl = pl.ds(j * CH, CH)
            cps.append(
                pltpu.async_copy(emb_hbm.at[idx_v.at[sl]], emb_v.at[sl], sem))
            cps.append(
                pltpu.async_copy(xyz_hbm.at[idx_v.at[sl]], xyz_v.at[sl], sem2))
        for cp in cps:
            cp.wait()

        def xform(v, carry):
            rvec = iota + v * L
            sl16 = pl.ds(v * L, L)
            x = plsc.load_gather(xyz_v, [rvec, c0i])
            y = plsc.load_gather(xyz_v, [rvec, c1i])
            z = plsc.load_gather(xyz_v, [rvec, c2i])
            xyzw_t[0, sl16] = x
            xyzw_t[1, sl16] = y
            xyzw_t[2, sl16] = z
            sx = x - cpx
            sy = y - cpy
            sz = z - cpz
            v0 = r[0] * sx + r[3] * sy + r[6] * sz
            v1 = r[1] * sx + r[4] * sy + r[7] * sz
            v2 = r[2] * sx + r[5] * sy + r[8] * sz
            den = v2 + 1e-9
            pers_t[0, sl16] = v0 / den
            pers_t[1, sl16] = v1 / den
            pers_t[2, sl16] = v2
            return carry

        lax.fori_loop(0, U // L, xform, 0)

        def tpose(q, carry):
            k = q >> 5
            t = (q >> 3) & 3
            fm = q & 7
            col = jnp.full((L,), t * 8 + fm, jnp.int32)
            row0 = k * 128
            dst = k * FT + t
            for j in range(8):
                g = plsc.load_gather(emb_v, [row0 + j * L + iota, col])
                feats_t[dst, pl.ds(fm * 128 + j * L, L)] = g
            return carry

        lax.fori_loop(0, 256, tpose, 0)

        outs = []
        for k in range(K):
            for t in range(FT):
                off = ((sr * K + k) * 128 + t * RT + c) * U
                outs.append(pltpu.async_copy(
                    feats_t.at[k * FT + t], feats_hbm.at[pl.ds(off, U)], sem3))
        for comp in range(3):
            off = ((sr * 3 + comp) * RT + c) * U
            outs.append(pltpu.async_copy(
                pers_t.at[comp], pers_hbm.at[pl.ds(off, U)], sem3))
            outs.append(pltpu.async_copy(
                xyzw_t.at[comp], xyzw_hbm.at[pl.ds(off, U)], sem3))
        for cp in outs:
            cp.wait()
        return carry

    lax.fori_loop(0, UPW, unit, 0)


@jax.jit
def _sc_gather(points_embeding, xyz_pad, pidx_tiles, cam):
    f32 = jnp.float32
    run = pl.kernel(
        _sc_body,
        out_type=(
            jax.ShapeDtypeStruct((M * FEAT,), f32),
            jax.ShapeDtypeStruct((M * 3,), f32),
            jax.ShapeDtypeStruct((M * 3,), f32),
        ),
        mesh=plsc.VectorSubcoreMesh(core_axis_name="c", subcore_axis_name="s"),
        compiler_params=pltpu.CompilerParams(
            needs_layout_passes=False, use_tc_tiling_on_sc=False),
        scratch_types=[
            pltpu.VMEM((U,), jnp.int32),
            pltpu.VMEM((U, FEAT), f32),
            pltpu.VMEM((U, XP), f32),
            pltpu.VMEM((K * FT, 128 * 8), f32),
            pltpu.VMEM((3, U), f32),
            pltpu.VMEM((3, U), f32),
            pltpu.VMEM((L,), f32),
            pltpu.SemaphoreType.DMA,
            pltpu.SemaphoreType.DMA,
            pltpu.SemaphoreType.DMA,
        ],
    )
    return run(points_embeding, xyz_pad, pidx_tiles, cam)


def kernel(xyz, points_embeding, camrotc2w, campos, sample_pidx):
    pidx_tiles = (sample_pidx.reshape(RT, 128, SR, K)
                  .transpose(2, 0, 3, 1).reshape(-1).astype(jnp.int32))
    cam = jnp.concatenate(
        [jnp.zeros((1,), jnp.float32), camrotc2w.reshape(9),
         campos.reshape(3), jnp.zeros((3,), jnp.float32)]).astype(jnp.float32)
    zcol = jnp.zeros((N,), jnp.float32)
    xyz_pad = jnp.stack(
        [xyz[:, 0], xyz[:, 1], xyz[:, 2], zcol, zcol, zcol, zcol, zcol],
        axis=1)
    embT = points_embeding.T
    tail = jnp.pad(points_embeding[NTF * 128:], ((0, 128 - NTAIL), (0, 0))).T
    emb_rm = _sc_transpose(embT, tail).reshape(N, FEAT)
    feats_img, pers_img, xyzw_img = _sc_gather(
        emb_rm, xyz_pad, pidx_tiles, cam)
    feats = (feats_img.reshape(SR, K, FT, RT, 8, 128)
             .transpose(3, 5, 0, 1, 2, 4).reshape(1, R, SR, K, FEAT))
    pers = (pers_img.reshape(SR, 3, RT, K, 128)
            .transpose(2, 4, 0, 3, 1).reshape(1, R, SR, K, 3))
    xyzw = (xyzw_img.reshape(SR, 3, RT, K, 128)
            .transpose(2, 4, 0, 3, 1).reshape(1, R, SR, K, 3))
    sample_pnt_mask = sample_pidx >= 0
    Rw2c = jnp.eye(3, dtype=xyz.dtype)
    return (feats, pers, xyzw, sample_pnt_mask, Rw2c)


# speedup vs baseline: 2.5359x; 1.0507x over previous
import functools

import jax
import jax.numpy as jnp
from jax import lax
from jax.experimental import pallas as pl
from jax.experimental.pallas import tpu as pltpu
from jax.experimental.pallas import tpu_sc as plsc

N = 500000
FEAT = 32
B, R, SR, K = 1, 4096, 24, 8
M = B * R * SR * K
NW = 32
U = 1024
RT = R // 128
NU = SR * RT
UPW = NU // NW
CH = 128
NCH = U // CH
L = 16
XP = 8
FT = FEAT // 8


def _bcast(cam_v, k):
    return plsc.load_gather(cam_v, [jnp.full((L,), k, jnp.int32)])


NTF = N // 128
NTAIL = N - NTF * 128


ST = 1024
NST = (NTF * 128) // ST
STPW = NST // NW
STX = NST - STPW * NW
LEFT = NTF - (NST * ST) // 128


def _tr_body(embT_hbm, tail_hbm, out_hbm, colbuf, rowbuf, semT):
    wid = lax.axis_index("s") * 2 + lax.axis_index("c")
    iota = lax.iota(jnp.int32, L)
    iota_lo = iota * ST
    iota_hi = iota_lo + L * ST

    def transp(npts):
        def inner(rloc, carry):
            g0 = plsc.load_gather(colbuf, [iota_lo + rloc])
            g1 = plsc.load_gather(colbuf, [iota_hi + rloc])
            rowbuf[pl.ds(rloc * FEAT, L)] = g0
            rowbuf[pl.ds(rloc * FEAT + L, L)] = g1
            return carry

        lax.fori_loop(0, npts, inner, 0, unroll=4)

    def block(col0, npts):
        cps = [pltpu.async_copy(
            embT_hbm.at[f, pl.ds(col0, npts)],
            colbuf.at[pl.ds(f * ST, npts)], semT)
            for f in range(FEAT)]
        for cp in cps:
            cp.wait()
        transp(npts)
        pltpu.sync_copy(rowbuf.at[pl.ds(0, npts * FEAT)],
                        out_hbm.at[pl.ds(col0 * FEAT, npts * FEAT)])

    def step(i, carry):
        block((wid + i * NW) * ST, ST)
        return carry

    lax.fori_loop(0, STPW, step, 0)

    @pl.when(wid < STX)
    def _():
        block((STPW * NW + wid) * ST, ST)

    @pl.when((wid >= STX) & (wid < STX + LEFT))
    def _():
        block(NST * ST + (wid - STX) * 128, 128)

    @pl.when(wid == NW - 1)
    def _():
        cps = [pltpu.async_copy(
            tail_hbm.at[f], colbuf.at[pl.ds(f * ST, 128)], semT)
            for f in range(FEAT)]
        for cp in cps:
            cp.wait()
        transp(NTAIL)
        pltpu.sync_copy(rowbuf.at[pl.ds(0, NTAIL * FEAT)],
                        out_hbm.at[pl.ds(NTF * 128 * FEAT, NTAIL * FEAT)])


@jax.jit
def _sc_transpose(embT, tail):
    run = pl.kernel(
        _tr_body,
        out_type=jax.ShapeDtypeStruct((N * FEAT,), jnp.float32),
        mesh=plsc.VectorSubcoreMesh(core_axis_name="c", subcore_axis_name="s"),
        compiler_params=pltpu.CompilerParams(
            needs_layout_passes=False, use_tc_tiling_on_sc=True),
        scratch_types=[
            pltpu.VMEM((FEAT * ST,), jnp.float32),
            pltpu.VMEM((ST * FEAT,), jnp.float32),
            pltpu.SemaphoreType.DMA,
        ],
    )
    return run(embT, tail)


def _sc_body(emb_hbm, xyz_hbm, pidx_hbm, cam_hbm,
             feats_hbm, pers_hbm, xyzw_hbm,
             idx_v2, emb_v2, xyz_v2, feats_t, pers_t, xyzw_t, cam_v,
             sem_e0, sem_e1, sem_x0, sem_x1, sem3):
    wid = lax.axis_index("s") * 2 + lax.axis_index("c")

    pltpu.sync_copy(cam_hbm, cam_v)
    r = [_bcast(cam_v, k + 1) for k in range(9)]
    cpx = _bcast(cam_v, 10)
    cpy = _bcast(cam_v, 11)
    cpz = _bcast(cam_v, 12)
    iota = lax.iota(jnp.int32, L)
    c0i = jnp.full((L,), 0, jnp.int32)
    c1i = jnp.full((L,), 1, jnp.int32)
    c2i = jnp.full((L,), 2, jnp.int32)

    bufs = [(idx_v2.at[0], emb_v2.at[0], xyz_v2.at[0], sem_e0, sem_x0),
            (idx_v2.at[1], emb_v2.at[1], xyz_v2.at[1], sem_e1, sem_x1)]

    def fire(u, b):
        idx_v, emb_v, xyz_v, sem_e, sem_x = bufs[b]
        pltpu.sync_copy(pidx_hbm.at[pl.ds(u * U, U)], idx_v)
        for j in range(NCH):
            sl = pl.ds(j * CH, CH)
            pltpu.async_copy(emb_hbm.at[idx_v.at[sl]], emb_v.at[sl], sem_e)
            pltpu.async_copy(xyz_hbm.at[idx_v.at[sl]], xyz_v.at[sl], sem_x)

    def drain(b):
        idx_v, emb_v, xyz_v, sem_e, sem_x = bufs[b]
        pltpu.make_async_copy(emb_hbm.at[pl.ds(0, U)], emb_v, sem_e).wait()
        pltpu.make_async_copy(xyz_hbm.at[pl.ds(0, U)], xyz_v, sem_x).wait()

    def compute(u, b):
        _, emb_v, xyz_v, _, _ = bufs[b]
        sr = u // RT
        c = u % RT

        def xform(v, carry):
            rvec = iota + v * L
            sl16 = pl.ds(v * L, L)
            x = plsc.load_gather(xyz_v, [rvec, c0i])
            y = plsc.load_gather(xyz_v, [rvec, c1i])
            z = plsc.load_gather(xyz_v, [rvec, c2i])
            xyzw_t[0, sl16] = x
            xyzw_t[1, sl16] = y
            xyzw_t[2, sl16] = z
            sx = x - cpx
            sy = y - cpy
            sz = z - cpz
            v0 = r[0] * sx + r[3] * sy + r[6] * sz
            v1 = r[1] * sx + r[4] * sy + r[7] * sz
            v2 = r[2] * sx + r[5] * sy + r[8] * sz
            den = v2 + 1e-9
            pers_t[0, sl16] = v0 / den
            pers_t[1, sl16] = v1 / den
            pers_t[2, sl16] = v2
            return carry

        lax.fori_loop(0, U // L, xform, 0)

        def tpose(q, carry):
            k = q >> 5
            t = (q >> 3) & 3
            fm = q & 7
            col = jnp.full((L,), t * 8 + fm, jnp.int32)
            row0 = k * 128
            dst = k * FT + t
            for j in range(8):
                g = plsc.load_gather(emb_v, [row0 + j * L + iota, col])
                feats_t[dst, pl.ds(fm * 128 + j * L, L)] = g
            return carry

        lax.fori_loop(0, 256, tpose, 0)

        outs = []
        for k in range(K):
            for t in range(FT):
                off = ((sr * K + k) * 128 + t * RT + c) * U
                outs.append(pltpu.async_copy(
                    feats_t.at[k * FT + t], feats_hbm.at[pl.ds(off, U)], sem3))
        for comp in range(3):
            off = ((sr * 3 + comp) * RT + c) * U
            outs.append(pltpu.async_copy(
                pers_t.at[comp], pers_hbm.at[pl.ds(off, U)], sem3))
            outs.append(pltpu.async_copy(
                xyzw_t.at[comp], xyzw_hbm.at[pl.ds(off, U)], sem3))
        for cp in outs:
            cp.wait()

    u0 = wid * UPW
    fire(u0, 0)
    fire(u0 + 1, 1)

    def step2(i, carry):
        i2 = i * 2
        for b in range(2):
            u = u0 + i2 + b
            drain(b)
            compute(u, b)

            @pl.when(i2 + b + 2 < UPW)
            def _():
                fire(u + 2, b)
        return carry

    lax.fori_loop(0, UPW // 2, step2, 0)


@jax.jit
def _sc_gather(points_embeding, xyz_pad, pidx_tiles, cam):
    f32 = jnp.float32
    run = pl.kernel(
        _sc_body,
        out_type=(
            jax.ShapeDtypeStruct((M * FEAT,), f32),
            jax.ShapeDtypeStruct((M * 3,), f32),
            jax.ShapeDtypeStruct((M * 3,), f32),
        ),
        mesh=plsc.VectorSubcoreMesh(core_axis_name="c", subcore_axis_name="s"),
        compiler_params=pltpu.CompilerParams(
            needs_layout_passes=False, use_tc_tiling_on_sc=False),
        scratch_types=[
            pltpu.VMEM((2, U), jnp.int32),
            pltpu.VMEM((2, U, FEAT), f32),
            pltpu.VMEM((2, U, XP), f32),
            pltpu.VMEM((K * FT, 128 * 8), f32),
            pltpu.VMEM((3, U), f32),
            pltpu.VMEM((3, U), f32),
            pltpu.VMEM((L,), f32),
            pltpu.SemaphoreType.DMA,
            pltpu.SemaphoreType.DMA,
            pltpu.SemaphoreType.DMA,
            pltpu.SemaphoreType.DMA,
            pltpu.SemaphoreType.DMA,
        ],
    )
    return run(points_embeding, xyz_pad, pidx_tiles, cam)


def kernel(xyz, points_embeding, camrotc2w, campos, sample_pidx):
    pidx_tiles = (sample_pidx.reshape(RT, 128, SR, K)
                  .transpose(2, 0, 3, 1).reshape(-1).astype(jnp.int32))
    cam = jnp.concatenate(
        [jnp.zeros((1,), jnp.float32), camrotc2w.reshape(9),
         campos.reshape(3), jnp.zeros((3,), jnp.float32)]).astype(jnp.float32)
    zcol = jnp.zeros((N,), jnp.float32)
    xyz_pad = jnp.stack(
        [xyz[:, 0], xyz[:, 1], xyz[:, 2], zcol, zcol, zcol, zcol, zcol],
        axis=1)
    embT = points_embeding.T
    tail = jnp.pad(points_embeding[NTF * 128:], ((0, 128 - NTAIL), (0, 0))).T
    emb_rm = _sc_transpose(embT, tail).reshape(N, FEAT)
    feats_img, pers_img, xyzw_img = _sc_gather(
        emb_rm, xyz_pad, pidx_tiles, cam)
    feats = (feats_img.reshape(SR, K, FT, RT, 8, 128)
             .transpose(3, 5, 0, 1, 2, 4).reshape(1, R, SR, K, FEAT))
    pers = (pers_img.reshape(SR, 3, RT, K, 128)
            .transpose(2, 4, 0, 3, 1).reshape(1, R, SR, K, 3))
    xyzw = (xyzw_img.reshape(SR, 3, RT, K, 128)
            .transpose(2, 4, 0, 3, 1).reshape(1, R, SR, K, 3))
    sample_pnt_mask = sample_pidx >= 0
    Rw2c = jnp.eye(3, dtype=xyz.dtype)
    return (feats, pers, xyzw, sample_pnt_mask, Rw2c)
